# Initial kernel scaffold; baseline (speedup 1.0000x reference)
import jax
import jax.numpy as jnp
from jax.experimental import pallas as pl


def kernel(node_features, W, M, edge_features, node_w1, node_b1, node_w2, node_b2, dir_w1, dir_b1, dir_w2, dir_b2, bi_w1, bi_b1, bi_w2, bi_b2, ln_g, ln_b):
    raise NotImplementedError("write your pallas kernel here")


# speedup vs baseline: 2.2888x; 2.2888x over previous
import jax
import jax.numpy as jnp
from jax.experimental import pallas as pl
from jax.experimental.pallas import tpu as pltpu

N = 512
D = 128
EDGE_DIM = 16
JB = 8
STEPS = N // JB
THR = 0.5


def _body(xj_ref, x_ref, Wb_ref, Wf_ref, Mtb_ref, Mtf_ref, ef_ref,
          w1aTd_ref, w1bTd_ref, w1eTd_ref, b1d_ref, w2Td_ref, b2d_ref,
          w1aTb_ref, w1bTb_ref, b1b_ref, w2Tb_ref, b2b_ref,
          nw1T_ref, nb1_ref, nw2T_ref, nb2_ref, lng_ref, lnb_ref,
          out_ref,
          Bd_ref, Bb_ref, Sd_ref, Sb_ref):
    jb = pl.program_id(0)

    @pl.when(jb == 0)
    def _init():
        x = x_ref[...]
        Bd_ref[...] = x @ w1bTd_ref[...] + b1d_ref[...]
        Bb_ref[...] = x @ w1bTb_ref[...] + b1b_ref[...]
        Sd_ref[...] = jnp.zeros((N, D), jnp.float32)
        Sb_ref[...] = jnp.zeros((N, D), jnp.float32)

    xj = xj_ref[...]

    Ad = xj @ w1aTd_ref[...]
    E = ef_ref[...] @ w1eTd_ref[...]
    pre_d = Ad[:, None, :] + Bd_ref[...][None, :, :] + E.reshape(JB, N, D)
    r_d = jnp.maximum(pre_d, 0.0)
    aW = jnp.abs(Wb_ref[...])
    wt_d = jnp.where(aW > THR, aW, 0.0)
    Sd_ref[...] += jnp.sum(wt_d[:, :, None] * r_d, axis=0)

    Ab = xj @ w1aTb_ref[...]
    pre_b = Ab[:, None, :] + Bb_ref[...][None, :, :]
    r_b = jnp.maximum(pre_b, 0.0)
    aM = jnp.abs(Mtb_ref[...])
    row_ids = jb * JB + jax.lax.broadcasted_iota(jnp.int32, (JB, N), 0)
    col_ids = jax.lax.broadcasted_iota(jnp.int32, (JB, N), 1)
    wt_b = jnp.where((aM > THR) & (row_ids != col_ids), aM, 0.0)
    Sb_ref[...] += jnp.sum(wt_b[:, :, None] * r_b, axis=0)

    @pl.when(jb == STEPS - 1)
    def _fin():
        x = x_ref[...]
        ones_col = jnp.ones((N, 1), jnp.float32)
        dn = (((0,), (0,)), ((), ()))
        aWf = jnp.abs(Wf_ref[...])
        wtf_d = jnp.where(aWf > THR, aWf, 0.0)
        sw_d = jax.lax.dot_general(wtf_d, ones_col, dn,
                                   preferred_element_type=jnp.float32)
        aMf = jnp.abs(Mtf_ref[...])
        rid = jax.lax.broadcasted_iota(jnp.int32, (N, N), 0)
        cid = jax.lax.broadcasted_iota(jnp.int32, (N, N), 1)
        wtf_b = jnp.where((aMf > THR) & (rid != cid), aMf, 0.0)
        sw_b = jax.lax.dot_general(wtf_b, ones_col, dn,
                                   preferred_element_type=jnp.float32)
        hd = Sd_ref[...] @ w2Td_ref[...] + sw_d * b2d_ref[...]
        hb = Sb_ref[...] @ w2Tb_ref[...] + sw_b * b2b_ref[...]
        hs = (jnp.maximum(x @ nw1T_ref[...] + nb1_ref[...], 0.0)
              @ nw2T_ref[...] + nb2_ref[...])
        h = hs + hd + hb
        mean = jnp.mean(h, axis=1, keepdims=True)
        c = h - mean
        var = jnp.mean(c * c, axis=1, keepdims=True)
        out_ref[...] = (c * jax.lax.rsqrt(var + 1e-5) * lng_ref[...]
                        + lnb_ref[...])


def kernel(node_features, W, M, edge_features, node_w1, node_b1, node_w2,
           node_b2, dir_w1, dir_b1, dir_w2, dir_b2, bi_w1, bi_b1, bi_w2,
           bi_b2, ln_g, ln_b):
    x = node_features
    Mt = M.T
    ef2 = edge_features.reshape(N * N, EDGE_DIM)
    r1 = lambda v: v.reshape(1, D)
    full = lambda shape: pl.BlockSpec(shape, lambda j: (0, 0))
    grid_spec = pltpu.PrefetchScalarGridSpec(
        num_scalar_prefetch=0,
        grid=(STEPS,),
        in_specs=[
            pl.BlockSpec((JB, D), lambda j: (j, 0)),
            full((N, D)),
            pl.BlockSpec((JB, N), lambda j: (j, 0)),
            full((N, N)),
            pl.BlockSpec((JB, N), lambda j: (j, 0)),
            full((N, N)),
            pl.BlockSpec((JB * N, EDGE_DIM), lambda j: (j, 0)),
            full((D, D)), full((D, D)), full((EDGE_DIM, D)),
            full((1, D)), full((D, D)), full((1, D)),
            full((D, D)), full((D, D)),
            full((1, D)), full((D, D)), full((1, D)),
            full((D, D)), full((1, D)), full((D, D)), full((1, D)),
            full((1, D)), full((1, D)),
        ],
        out_specs=pl.BlockSpec((N, D), lambda j: (0, 0)),
        scratch_shapes=[pltpu.VMEM((N, D), jnp.float32)] * 4,
    )
    out = pl.pallas_call(
        _body,
        grid_spec=grid_spec,
        out_shape=jax.ShapeDtypeStruct((N, D), jnp.float32),
    )(x, x, W, W, Mt, Mt, ef2,
      dir_w1[:, :D].T, dir_w1[:, D:2 * D].T, dir_w1[:, 2 * D:].T,
      r1(dir_b1), dir_w2.T, r1(dir_b2),
      bi_w1[:, :D].T, bi_w1[:, D:].T, r1(bi_b1), bi_w2.T, r1(bi_b2),
      node_w1.T, r1(node_b1), node_w2.T, r1(node_b2),
      r1(ln_g), r1(ln_b))
    return out


# speedup vs baseline: 2.5082x; 1.0959x over previous
import jax
import jax.numpy as jnp
from jax.experimental import pallas as pl
from jax.experimental.pallas import tpu as pltpu

N = 512
D = 128
EDGE_DIM = 16
JB = 16
STEPS = N // JB
THR = 0.5


def _body(xj_ref, x_ref, Wb_ref, Wf_ref, Mtb_ref, Mtf_ref, ef_ref,
          w1aTd_ref, w1bTd_ref, w1eTd_ref, b1d_ref, w2Td_ref, b2d_ref,
          w1aTb_ref, w1bTb_ref, b1b_ref, w2Tb_ref, b2b_ref,
          nw1T_ref, nb1_ref, nw2T_ref, nb2_ref, lng_ref, lnb_ref,
          out_ref,
          Bd_ref, Bb_ref, Sd_ref, Sb_ref):
    jb = pl.program_id(0)

    @pl.when(jb == 0)
    def _init():
        x = x_ref[...]
        Bd_ref[...] = x @ w1bTd_ref[...] + b1d_ref[...]
        Bb_ref[...] = x @ w1bTb_ref[...] + b1b_ref[...]
        Sd_ref[...] = jnp.zeros((N, D), jnp.float32)
        Sb_ref[...] = jnp.zeros((N, D), jnp.float32)

    xj = xj_ref[...]

    Ad = xj @ w1aTd_ref[...]
    E = ef_ref[...] @ w1eTd_ref[...]
    pre_d = Ad[:, None, :] + Bd_ref[...][None, :, :] + E.reshape(JB, N, D)
    r_d = jnp.maximum(pre_d, 0.0)
    aW = jnp.abs(Wb_ref[...])
    wt_d = jnp.where(aW > THR, aW, 0.0)
    Sd_ref[...] += jnp.sum(wt_d[:, :, None] * r_d, axis=0)

    Ab = xj @ w1aTb_ref[...]
    pre_b = Ab[:, None, :] + Bb_ref[...][None, :, :]
    r_b = jnp.maximum(pre_b, 0.0)
    aM = jnp.abs(Mtb_ref[...])
    row_ids = jb * JB + jax.lax.broadcasted_iota(jnp.int32, (JB, N), 0)
    col_ids = jax.lax.broadcasted_iota(jnp.int32, (JB, N), 1)
    wt_b = jnp.where((aM > THR) & (row_ids != col_ids), aM, 0.0)
    Sb_ref[...] += jnp.sum(wt_b[:, :, None] * r_b, axis=0)

    @pl.when(jb == STEPS - 1)
    def _fin():
        x = x_ref[...]
        ones_col = jnp.ones((N, 1), jnp.float32)
        dn = (((0,), (0,)), ((), ()))
        aWf = jnp.abs(Wf_ref[...])
        wtf_d = jnp.where(aWf > THR, aWf, 0.0)
        sw_d = jax.lax.dot_general(wtf_d, ones_col, dn,
                                   preferred_element_type=jnp.float32)
        aMf = jnp.abs(Mtf_ref[...])
        rid = jax.lax.broadcasted_iota(jnp.int32, (N, N), 0)
        cid = jax.lax.broadcasted_iota(jnp.int32, (N, N), 1)
        wtf_b = jnp.where((aMf > THR) & (rid != cid), aMf, 0.0)
        sw_b = jax.lax.dot_general(wtf_b, ones_col, dn,
                                   preferred_element_type=jnp.float32)
        hd = Sd_ref[...] @ w2Td_ref[...] + sw_d * b2d_ref[...]
        hb = Sb_ref[...] @ w2Tb_ref[...] + sw_b * b2b_ref[...]
        hs = (jnp.maximum(x @ nw1T_ref[...] + nb1_ref[...], 0.0)
              @ nw2T_ref[...] + nb2_ref[...])
        h = hs + hd + hb
        mean = jnp.mean(h, axis=1, keepdims=True)
        c = h - mean
        var = jnp.mean(c * c, axis=1, keepdims=True)
        out_ref[...] = (c * jax.lax.rsqrt(var + 1e-5) * lng_ref[...]
                        + lnb_ref[...])


def kernel(node_features, W, M, edge_features, node_w1, node_b1, node_w2,
           node_b2, dir_w1, dir_b1, dir_w2, dir_b2, bi_w1, bi_b1, bi_w2,
           bi_b2, ln_g, ln_b):
    x = node_features
    Mt = M.T
    ef2 = edge_features.reshape(N * N, EDGE_DIM)
    r1 = lambda v: v.reshape(1, D)
    full = lambda shape: pl.BlockSpec(shape, lambda j: (0, 0))
    grid_spec = pltpu.PrefetchScalarGridSpec(
        num_scalar_prefetch=0,
        grid=(STEPS,),
        in_specs=[
            pl.BlockSpec((JB, D), lambda j: (j, 0)),
            full((N, D)),
            pl.BlockSpec((JB, N), lambda j: (j, 0)),
            full((N, N)),
            pl.BlockSpec((JB, N), lambda j: (j, 0)),
            full((N, N)),
            pl.BlockSpec((JB * N, EDGE_DIM), lambda j: (j, 0)),
            full((D, D)), full((D, D)), full((EDGE_DIM, D)),
            full((1, D)), full((D, D)), full((1, D)),
            full((D, D)), full((D, D)),
            full((1, D)), full((D, D)), full((1, D)),
            full((D, D)), full((1, D)), full((D, D)), full((1, D)),
            full((1, D)), full((1, D)),
        ],
        out_specs=pl.BlockSpec((N, D), lambda j: (0, 0)),
        scratch_shapes=[pltpu.VMEM((N, D), jnp.float32)] * 4,
    )
    out = pl.pallas_call(
        _body,
        grid_spec=grid_spec,
        out_shape=jax.ShapeDtypeStruct((N, D), jnp.float32),
    )(x, x, W, W, Mt, Mt, ef2,
      dir_w1[:, :D].T, dir_w1[:, D:2 * D].T, dir_w1[:, 2 * D:].T,
      r1(dir_b1), dir_w2.T, r1(dir_b2),
      bi_w1[:, :D].T, bi_w1[:, D:].T, r1(bi_b1), bi_w2.T, r1(bi_b2),
      node_w1.T, r1(node_b1), node_w2.T, r1(node_b2),
      r1(ln_g), r1(ln_b))
    return out


# speedup vs baseline: 2.5306x; 1.0089x over previous
import jax
import jax.numpy as jnp
from jax.experimental import pallas as pl
from jax.experimental.pallas import tpu as pltpu

N = 512
D = 128
EDGE_DIM = 16
JB = 32
STEPS = N // JB
THR = 0.5


def _body(xj_ref, x_ref, Wb_ref, Wf_ref, Mtb_ref, Mtf_ref, ef_ref,
          w1aTd_ref, w1bTd_ref, w1eTd_ref, b1d_ref, w2Td_ref, b2d_ref,
          w1aTb_ref, w1bTb_ref, b1b_ref, w2Tb_ref, b2b_ref,
          nw1T_ref, nb1_ref, nw2T_ref, nb2_ref, lng_ref, lnb_ref,
          out_ref,
          Bd_ref, Bb_ref, Sd_ref, Sb_ref):
    jb = pl.program_id(0)

    @pl.when(jb == 0)
    def _init():
        x = x_ref[...]
        Bd_ref[...] = x @ w1bTd_ref[...] + b1d_ref[...]
        Bb_ref[...] = x @ w1bTb_ref[...] + b1b_ref[...]
        Sd_ref[...] = jnp.zeros((N, D), jnp.float32)
        Sb_ref[...] = jnp.zeros((N, D), jnp.float32)

    xj = xj_ref[...]

    Ad = xj @ w1aTd_ref[...]
    E = ef_ref[...] @ w1eTd_ref[...]
    pre_d = Ad[:, None, :] + Bd_ref[...][None, :, :] + E.reshape(JB, N, D)
    r_d = jnp.maximum(pre_d, 0.0)
    aW = jnp.abs(Wb_ref[...])
    wt_d = jnp.where(aW > THR, aW, 0.0)
    Sd_ref[...] += jnp.sum(wt_d[:, :, None] * r_d, axis=0)

    Ab = xj @ w1aTb_ref[...]
    pre_b = Ab[:, None, :] + Bb_ref[...][None, :, :]
    r_b = jnp.maximum(pre_b, 0.0)
    aM = jnp.abs(Mtb_ref[...])
    row_ids = jb * JB + jax.lax.broadcasted_iota(jnp.int32, (JB, N), 0)
    col_ids = jax.lax.broadcasted_iota(jnp.int32, (JB, N), 1)
    wt_b = jnp.where((aM > THR) & (row_ids != col_ids), aM, 0.0)
    Sb_ref[...] += jnp.sum(wt_b[:, :, None] * r_b, axis=0)

    @pl.when(jb == STEPS - 1)
    def _fin():
        x = x_ref[...]
        ones_col = jnp.ones((N, 1), jnp.float32)
        dn = (((0,), (0,)), ((), ()))
        aWf = jnp.abs(Wf_ref[...])
        wtf_d = jnp.where(aWf > THR, aWf, 0.0)
        sw_d = jax.lax.dot_general(wtf_d, ones_col, dn,
                                   preferred_element_type=jnp.float32)
        aMf = jnp.abs(Mtf_ref[...])
        rid = jax.lax.broadcasted_iota(jnp.int32, (N, N), 0)
        cid = jax.lax.broadcasted_iota(jnp.int32, (N, N), 1)
        wtf_b = jnp.where((aMf > THR) & (rid != cid), aMf, 0.0)
        sw_b = jax.lax.dot_general(wtf_b, ones_col, dn,
                                   preferred_element_type=jnp.float32)
        hd = Sd_ref[...] @ w2Td_ref[...] + sw_d * b2d_ref[...]
        hb = Sb_ref[...] @ w2Tb_ref[...] + sw_b * b2b_ref[...]
        hs = (jnp.maximum(x @ nw1T_ref[...] + nb1_ref[...], 0.0)
              @ nw2T_ref[...] + nb2_ref[...])
        h = hs + hd + hb
        mean = jnp.mean(h, axis=1, keepdims=True)
        c = h - mean
        var = jnp.mean(c * c, axis=1, keepdims=True)
        out_ref[...] = (c * jax.lax.rsqrt(var + 1e-5) * lng_ref[...]
                        + lnb_ref[...])


def kernel(node_features, W, M, edge_features, node_w1, node_b1, node_w2,
           node_b2, dir_w1, dir_b1, dir_w2, dir_b2, bi_w1, bi_b1, bi_w2,
           bi_b2, ln_g, ln_b):
    x = node_features
    Mt = M.T
    ef2 = edge_features.reshape(N * N, EDGE_DIM)
    r1 = lambda v: v.reshape(1, D)
    full = lambda shape: pl.BlockSpec(shape, lambda j: (0, 0))
    grid_spec = pltpu.PrefetchScalarGridSpec(
        num_scalar_prefetch=0,
        grid=(STEPS,),
        in_specs=[
            pl.BlockSpec((JB, D), lambda j: (j, 0)),
            full((N, D)),
            pl.BlockSpec((JB, N), lambda j: (j, 0)),
            full((N, N)),
            pl.BlockSpec((JB, N), lambda j: (j, 0)),
            full((N, N)),
            pl.BlockSpec((JB * N, EDGE_DIM), lambda j: (j, 0)),
            full((D, D)), full((D, D)), full((EDGE_DIM, D)),
            full((1, D)), full((D, D)), full((1, D)),
            full((D, D)), full((D, D)),
            full((1, D)), full((D, D)), full((1, D)),
            full((D, D)), full((1, D)), full((D, D)), full((1, D)),
            full((1, D)), full((1, D)),
        ],
        out_specs=pl.BlockSpec((N, D), lambda j: (0, 0)),
        scratch_shapes=[pltpu.VMEM((N, D), jnp.float32)] * 4,
    )
    out = pl.pallas_call(
        _body,
        grid_spec=grid_spec,
        out_shape=jax.ShapeDtypeStruct((N, D), jnp.float32),
    )(x, x, W, W, Mt, Mt, ef2,
      dir_w1[:, :D].T, dir_w1[:, D:2 * D].T, dir_w1[:, 2 * D:].T,
      r1(dir_b1), dir_w2.T, r1(dir_b2),
      bi_w1[:, :D].T, bi_w1[:, D:].T, r1(bi_b1), bi_w2.T, r1(bi_b2),
      node_w1.T, r1(node_b1), node_w2.T, r1(node_b2),
      r1(ln_g), r1(ln_b))
    return out


# speedup vs baseline: 2.5962x; 1.0259x over previous
import jax
import jax.numpy as jnp
from jax.experimental import pallas as pl
from jax.experimental.pallas import tpu as pltpu

N = 512
D = 128
EDGE_DIM = 16
JB = 32
STEPS = N // JB
THR = 0.5


def _body(xj_ref, x_ref, Wb_ref, Wf_ref, Mtb_ref, Mtf_ref, ef_ref,
          w1aTd_ref, w1bTd_ref, w1eTd_ref, b1d_ref, w2Td_ref, b2d_ref,
          w1aTb_ref, w1bTb_ref, b1b_ref, w2Tb_ref, b2b_ref,
          nw1T_ref, nb1_ref, nw2T_ref, nb2_ref, lng_ref, lnb_ref,
          out_ref,
          Bd_ref, Bb_ref, Sd_ref, Sb_ref):
    jb = pl.program_id(0)

    @pl.when(jb == 0)
    def _init():
        x = x_ref[...]
        Bd_ref[...] = x @ w1bTd_ref[...] + b1d_ref[...]
        Bb_ref[...] = x @ w1bTb_ref[...] + b1b_ref[...]
        Sd_ref[...] = jnp.zeros((N, D), jnp.float32)
        Sb_ref[...] = jnp.zeros((N, D), jnp.float32)

    xj = xj_ref[...]
    E = ef_ref[...] @ w1eTd_ref[...]

    Ab = xj @ w1aTb_ref[...]
    pre_b = Ab[:, None, :] + Bb_ref[...][None, :, :]
    r_b = jnp.maximum(pre_b, 0.0)
    aM = jnp.abs(Mtb_ref[...])
    row_ids = jb * JB + jax.lax.broadcasted_iota(jnp.int32, (JB, N), 0)
    col_ids = jax.lax.broadcasted_iota(jnp.int32, (JB, N), 1)
    wt_b = jnp.where((aM > THR) & (row_ids != col_ids), aM, 0.0)
    Sb_ref[...] += jnp.sum(wt_b[:, :, None] * r_b, axis=0)

    Ad = xj @ w1aTd_ref[...]
    pre_d = Ad[:, None, :] + Bd_ref[...][None, :, :] + E.reshape(JB, N, D)
    r_d = jnp.maximum(pre_d, 0.0)
    aW = jnp.abs(Wb_ref[...])
    wt_d = jnp.where(aW > THR, aW, 0.0)
    Sd_ref[...] += jnp.sum(wt_d[:, :, None] * r_d, axis=0)

    @pl.when(jb == STEPS - 1)
    def _fin():
        x = x_ref[...]
        ones_col = jnp.ones((N, 1), jnp.float32)
        dn = (((0,), (0,)), ((), ()))
        aWf = jnp.abs(Wf_ref[...])
        wtf_d = jnp.where(aWf > THR, aWf, 0.0)
        sw_d = jax.lax.dot_general(wtf_d, ones_col, dn,
                                   preferred_element_type=jnp.float32)
        aMf = jnp.abs(Mtf_ref[...])
        rid = jax.lax.broadcasted_iota(jnp.int32, (N, N), 0)
        cid = jax.lax.broadcasted_iota(jnp.int32, (N, N), 1)
        wtf_b = jnp.where((aMf > THR) & (rid != cid), aMf, 0.0)
        sw_b = jax.lax.dot_general(wtf_b, ones_col, dn,
                                   preferred_element_type=jnp.float32)
        hd = Sd_ref[...] @ w2Td_ref[...] + sw_d * b2d_ref[...]
        hb = Sb_ref[...] @ w2Tb_ref[...] + sw_b * b2b_ref[...]
        hs = (jnp.maximum(x @ nw1T_ref[...] + nb1_ref[...], 0.0)
              @ nw2T_ref[...] + nb2_ref[...])
        h = hs + hd + hb
        mean = jnp.mean(h, axis=1, keepdims=True)
        c = h - mean
        var = jnp.mean(c * c, axis=1, keepdims=True)
        out_ref[...] = (c * jax.lax.rsqrt(var + 1e-5) * lng_ref[...]
                        + lnb_ref[...])


def kernel(node_features, W, M, edge_features, node_w1, node_b1, node_w2,
           node_b2, dir_w1, dir_b1, dir_w2, dir_b2, bi_w1, bi_b1, bi_w2,
           bi_b2, ln_g, ln_b):
    x = node_features
    Mt = M.T
    ef2 = edge_features.reshape(N * N, EDGE_DIM)
    r1 = lambda v: v.reshape(1, D)
    full = lambda shape: pl.BlockSpec(shape, lambda j: (0, 0))
    grid_spec = pltpu.PrefetchScalarGridSpec(
        num_scalar_prefetch=0,
        grid=(STEPS,),
        in_specs=[
            pl.BlockSpec((JB, D), lambda j: (j, 0)),
            full((N, D)),
            pl.BlockSpec((JB, N), lambda j: (j, 0)),
            full((N, N)),
            pl.BlockSpec((JB, N), lambda j: (j, 0)),
            full((N, N)),
            pl.BlockSpec((JB * N, EDGE_DIM), lambda j: (j, 0)),
            full((D, D)), full((D, D)), full((EDGE_DIM, D)),
            full((1, D)), full((D, D)), full((1, D)),
            full((D, D)), full((D, D)),
            full((1, D)), full((D, D)), full((1, D)),
            full((D, D)), full((1, D)), full((D, D)), full((1, D)),
            full((1, D)), full((1, D)),
        ],
        out_specs=pl.BlockSpec((N, D), lambda j: (0, 0)),
        scratch_shapes=[pltpu.VMEM((N, D), jnp.float32)] * 4,
    )
    out = pl.pallas_call(
        _body,
        grid_spec=grid_spec,
        out_shape=jax.ShapeDtypeStruct((N, D), jnp.float32),
    )(x, x, W, W, Mt, Mt, ef2,
      dir_w1[:, :D].T, dir_w1[:, D:2 * D].T, dir_w1[:, 2 * D:].T,
      r1(dir_b1), dir_w2.T, r1(dir_b2),
      bi_w1[:, :D].T, bi_w1[:, D:].T, r1(bi_b1), bi_w2.T, r1(bi_b2),
      node_w1.T, r1(node_b1), node_w2.T, r1(node_b2),
      r1(ln_g), r1(ln_b))
    return out
